# baseline (speedup 1.0000x reference)
import functools

import jax
import jax.numpy as jnp
from jax import lax
from jax.experimental import pallas as pl
from jax.experimental.pallas import tpu as pltpu
from jax.experimental.pallas import tpu_sc as plsc

NC = 2
NS = 16
NW = NC * NS
LANES = 16
CH = 160
NBUF = 3


def _tree_max(vals):
    while len(vals) > 1:
        nxt = [jnp.maximum(vals[i], vals[i + 1]) for i in range(0, len(vals) - 1, 2)]
        if len(vals) % 2:
            nxt.append(vals[-1])
        vals = nxt
    return vals[0]


def _make_segmax(n_rows, num_segments, d):
    ncg = d // LANES
    n_chunks = n_rows // CH
    assert n_chunks * CH == n_rows
    max_my = -(-n_chunks // NW)
    pre_rows = max_my * CH
    chw = CH * d
    mesh = plsc.VectorSubcoreMesh(core_axis_name="c", subcore_axis_name="s",
                                  num_cores=NC, num_subcores=NS)

    def body(x_hbm, b_hbm, out_hbm, acc, xbuf, bbuf, semx, semb):
        wid = lax.axis_index("s") * NC + lax.axis_index("c")

        start_c = (wid * n_chunks) // NW
        end_c = ((wid + 1) * n_chunks) // NW
        n_my = end_c - start_c

        def start_fetch(i, slot):
            c = start_c + i
            pltpu.async_copy(x_hbm.at[pl.ds(c * chw, chw)],
                             xbuf.at[pl.ds(slot * chw, chw)], semx.at[slot])

        pltpu.async_copy(b_hbm.at[pl.ds(start_c * CH, pre_rows)], bbuf, semb)

        @pl.when(n_my > 0)
        def _prime0():
            start_fetch(0, 0)

        @pl.when(n_my > 1)
        def _prime1():
            start_fetch(1, 1)

        neg = jnp.full((LANES,), -jnp.inf, jnp.float32)
        def init_body(i, carry):
            for c in range(ncg):
                acc[pl.ds(i * d + c * LANES, LANES)] = neg
            return carry
        lax.fori_loop(0, num_segments, init_body, 0)

        pltpu.make_async_copy(b_hbm.at[pl.ds(0, pre_rows)], bbuf, semb).wait()

        def do_rowgroup(par, i, g):
            base = g * LANES
            rb = par * chw + base * d
            bvec = bbuf[pl.ds(i * CH + base, LANES)]
            s0 = bvec[0]
            s15 = bvec[LANES - 1]

            @pl.when(s0 == s15)
            def _fast():
                ab = s0 * d
                for c in range(ncg):
                    co = c * LANES
                    rows = [xbuf[pl.ds(rb + r * d + co, LANES)]
                            for r in range(LANES)]
                    m = _tree_max(rows)
                    aa = pl.ds(ab + co, LANES)
                    acc[aa] = jnp.maximum(acc[aa], m)

            @pl.when(s0 != s15)
            def _mixed():
                for j in range(LANES):
                    ab = bvec[j] * d
                    for c in range(ncg):
                        co = c * LANES
                        v = xbuf[pl.ds(rb + j * d + co, LANES)]
                        aa = pl.ds(ab + co, LANES)
                        acc[aa] = jnp.maximum(acc[aa], v)

        def chunk_body(i, carry):
            cur = lax.rem(i, NBUF)
            pltpu.make_async_copy(x_hbm.at[pl.ds(0, chw)],
                                  xbuf.at[pl.ds(cur * chw, chw)],
                                  semx.at[cur]).wait()

            @pl.when(i + 2 < n_my)
            def _prefetch():
                start_fetch(i + 2, lax.rem(i + 2, NBUF))

            def g_body(g, carry2):
                do_rowgroup(cur, i, g)
                return carry2
            lax.fori_loop(0, CH // LANES, g_body, 0)
            return carry
        lax.fori_loop(0, n_my, chunk_body, 0)

        pltpu.sync_copy(acc, out_hbm.at[wid])

    return pl.kernel(
        body,
        out_type=jax.ShapeDtypeStruct((NW, num_segments * d), jnp.float32),
        mesh=mesh,
        compiler_params=pltpu.CompilerParams(needs_layout_passes=False),
        scratch_types=[
            pltpu.VMEM((num_segments * d,), jnp.float32),
            pltpu.VMEM((NBUF * CH * d,), jnp.float32),
            pltpu.VMEM((pre_rows,), jnp.int32),
            pltpu.SemaphoreType.DMA((NBUF,)),
            pltpu.SemaphoreType.DMA,
        ],
    )


def _make_mlp(num_segments, d):
    def mlp_body(parts_ref, u_ref, w1_ref, b1_ref, w2_ref, b2_ref, out_ref):
        agg = parts_ref[0]
        for i in range(1, NW):
            agg = jnp.maximum(agg, parts_ref[i])
        agg = jnp.where(jnp.isfinite(agg), agg, 0.0)
        w1 = w1_ref[...]
        h = (jnp.dot(u_ref[...], w1[:d], preferred_element_type=jnp.float32)
             + jnp.dot(agg, w1[d:], preferred_element_type=jnp.float32)
             + b1_ref[...])
        h = jnp.maximum(h, 0.0)
        out_ref[...] = (jnp.dot(h, w2_ref[...],
                                preferred_element_type=jnp.float32)
                        + b2_ref[...])

    return pl.pallas_call(
        mlp_body,
        out_shape=jax.ShapeDtypeStruct((num_segments, d), jnp.float32),
    )


@jax.jit
def kernel(x, batch, u, W1, b1, W2, b2):
    n, d = x.shape
    num_segments = u.shape[0]
    batch = batch.astype(jnp.int32)
    parts = _make_segmax(n, num_segments, d)(x.reshape(n * d), batch)
    parts = parts.reshape(NW, num_segments, d)
    return _make_mlp(num_segments, d)(parts, u, W1, b1.reshape(1, d), W2,
                                      b2.reshape(1, d))


# speedup vs baseline: 1.1624x; 1.1624x over previous
import functools

import jax
import jax.numpy as jnp
from jax import lax
from jax.experimental import pallas as pl
from jax.experimental.pallas import tpu as pltpu
from jax.experimental.pallas import tpu_sc as plsc

NC = 2
NS = 16
NW = NC * NS
LANES = 16
CH = 160
NBUF = 3


def _tree_max(vals):
    while len(vals) > 1:
        nxt = [jnp.maximum(vals[i], vals[i + 1]) for i in range(0, len(vals) - 1, 2)]
        if len(vals) % 2:
            nxt.append(vals[-1])
        vals = nxt
    return vals[0]


def _make_segmax(n_rows, num_segments, d):
    ncg = d // LANES
    n_chunks = n_rows // CH
    assert n_chunks * CH == n_rows
    max_my = -(-n_chunks // NW)
    pre_rows = max_my * CH
    mesh = plsc.VectorSubcoreMesh(core_axis_name="c", subcore_axis_name="s",
                                  num_cores=NC, num_subcores=NS)

    def body(x_hbm, b_hbm, out_hbm, acc, xbuf, bbuf, semx, semb):
        wid = lax.axis_index("s") * NC + lax.axis_index("c")

        start_c = (wid * n_chunks) // NW
        end_c = ((wid + 1) * n_chunks) // NW
        n_my = end_c - start_c

        def start_fetch(i, slot):
            c = start_c + i
            pltpu.async_copy(x_hbm.at[pl.ds(c * CH, CH)], xbuf.at[slot],
                             semx.at[slot])

        pltpu.async_copy(b_hbm.at[pl.ds(start_c * CH, pre_rows)], bbuf, semb)

        @pl.when(n_my > 0)
        def _prime0():
            start_fetch(0, 0)

        @pl.when(n_my > 1)
        def _prime1():
            start_fetch(1, 1)

        neg = jnp.full((LANES,), -jnp.inf, jnp.float32)
        def init_body(i, carry):
            for c in range(ncg):
                acc[i, pl.ds(c * LANES, LANES)] = neg
            return carry
        lax.fori_loop(0, num_segments, init_body, 0)

        pltpu.make_async_copy(b_hbm.at[pl.ds(0, pre_rows)], bbuf, semb).wait()

        def do_rowgroup(par, i, g):
            base = g * LANES
            bvec = bbuf[pl.ds(i * CH + base, LANES)]
            s0 = bvec[0]
            s15 = bvec[LANES - 1]

            @pl.when(s0 == s15)
            def _fast():
                for c in range(ncg):
                    cs = pl.ds(c * LANES, LANES)
                    rows = [xbuf[par, base + r, cs] for r in range(LANES)]
                    m = _tree_max(rows)
                    acc[s0, cs] = jnp.maximum(acc[s0, cs], m)

            @pl.when(s0 != s15)
            def _mixed():
                for j in range(LANES):
                    sj = bvec[j]
                    for c in range(ncg):
                        cs = pl.ds(c * LANES, LANES)
                        v = xbuf[par, base + j, cs]
                        acc[sj, cs] = jnp.maximum(acc[sj, cs], v)

        def chunk_body(i, carry):
            cur = lax.rem(i, NBUF)
            pltpu.make_async_copy(x_hbm.at[pl.ds(0, CH)], xbuf.at[cur],
                                  semx.at[cur]).wait()

            @pl.when(i + 2 < n_my)
            def _prefetch():
                start_fetch(i + 2, lax.rem(i + 2, NBUF))

            def g_body(g, carry2):
                do_rowgroup(cur, i, g)
                return carry2
            lax.fori_loop(0, CH // LANES, g_body, 0)
            return carry
        lax.fori_loop(0, n_my, chunk_body, 0)

        pltpu.sync_copy(acc, out_hbm.at[wid])

    return pl.kernel(
        body,
        out_type=jax.ShapeDtypeStruct((NW, num_segments, d), jnp.float32),
        mesh=mesh,
        compiler_params=pltpu.CompilerParams(needs_layout_passes=False),
        scratch_types=[
            pltpu.VMEM((num_segments, d), jnp.float32),
            pltpu.VMEM((NBUF, CH, d), jnp.float32),
            pltpu.VMEM((pre_rows,), jnp.int32),
            pltpu.SemaphoreType.DMA((NBUF,)),
            pltpu.SemaphoreType.DMA,
        ],
    )


def _make_mlp(num_segments, d):
    def mlp_body(parts_ref, u_ref, w1_ref, b1_ref, w2_ref, b2_ref, out_ref):
        agg = parts_ref[0]
        for i in range(1, NW):
            agg = jnp.maximum(agg, parts_ref[i])
        agg = jnp.where(jnp.isfinite(agg), agg, 0.0)
        w1 = w1_ref[...]
        h = (jnp.dot(u_ref[...], w1[:d], preferred_element_type=jnp.float32)
             + jnp.dot(agg, w1[d:], preferred_element_type=jnp.float32)
             + b1_ref[...])
        h = jnp.maximum(h, 0.0)
        out_ref[...] = (jnp.dot(h, w2_ref[...],
                                preferred_element_type=jnp.float32)
                        + b2_ref[...])

    return pl.pallas_call(
        mlp_body,
        out_shape=jax.ShapeDtypeStruct((num_segments, d), jnp.float32),
    )


@jax.jit
def kernel(x, batch, u, W1, b1, W2, b2):
    n, d = x.shape
    num_segments = u.shape[0]
    batch = batch.astype(jnp.int32)
    parts = _make_segmax(n, num_segments, d)(x, batch)
    return _make_mlp(num_segments, d)(parts, u, W1, b1.reshape(1, d), W2,
                                      b2.reshape(1, d))


# speedup vs baseline: 1.2084x; 1.0396x over previous
import functools

import jax
import jax.numpy as jnp
from jax import lax
from jax.experimental import pallas as pl
from jax.experimental.pallas import tpu as pltpu
from jax.experimental.pallas import tpu_sc as plsc

NC = 2
NS = 16
NW = NC * NS
LANES = 16
CH = 160
NBUF = 3


def _tree_max(vals):
    while len(vals) > 1:
        nxt = [jnp.maximum(vals[i], vals[i + 1]) for i in range(0, len(vals) - 1, 2)]
        if len(vals) % 2:
            nxt.append(vals[-1])
        vals = nxt
    return vals[0]


def _make_segmax(n_rows, num_segments, d):
    ncg = d // LANES
    n_chunks = n_rows // CH
    assert n_chunks * CH == n_rows
    max_my = -(-n_chunks // NW)
    pre_rows = max_my * CH
    mesh = plsc.VectorSubcoreMesh(core_axis_name="c", subcore_axis_name="s",
                                  num_cores=NC, num_subcores=NS)

    def body(x_hbm, b_hbm, out_hbm, acc, xbuf, bbuf, semx, semb):
        wid = lax.axis_index("s") * NC + lax.axis_index("c")

        start_c = (wid * n_chunks) // NW
        end_c = ((wid + 1) * n_chunks) // NW
        n_my = end_c - start_c

        def start_fetch(i, slot):
            c = start_c + i
            pltpu.async_copy(x_hbm.at[pl.ds(c * CH, CH)], xbuf.at[slot],
                             semx.at[slot])

        pltpu.async_copy(b_hbm.at[pl.ds(start_c * CH, pre_rows)], bbuf, semb)

        @pl.when(n_my > 0)
        def _prime0():
            start_fetch(0, 0)

        @pl.when(n_my > 1)
        def _prime1():
            start_fetch(1, 1)

        neg = jnp.full((LANES,), -jnp.inf, jnp.float32)
        def init_body(i, carry):
            for c in range(ncg):
                acc[i, pl.ds(c * LANES, LANES)] = neg
            return carry
        lax.fori_loop(0, num_segments, init_body, 0)

        pltpu.make_async_copy(b_hbm.at[pl.ds(0, pre_rows)], bbuf, semb).wait()

        def do_rowgroup(par, i, g):
            base = g * LANES
            bvec = bbuf[pl.ds(i * CH + base, LANES)]
            s0 = bvec[0]
            s15 = bvec[LANES - 1]

            @pl.when(s0 == s15)
            def _fast():
                for c in range(ncg):
                    cs = pl.ds(c * LANES, LANES)
                    m = xbuf[par, base, cs]
                    for r in range(1, LANES):
                        m = jnp.maximum(m, xbuf[par, base + r, cs])
                    acc[s0, cs] = jnp.maximum(acc[s0, cs], m)

            @pl.when(s0 != s15)
            def _mixed():
                for j in range(LANES):
                    sj = bvec[j]
                    for c in range(ncg):
                        cs = pl.ds(c * LANES, LANES)
                        v = xbuf[par, base + j, cs]
                        acc[sj, cs] = jnp.maximum(acc[sj, cs], v)

        def chunk_body(i, carry):
            cur = lax.rem(i, NBUF)
            pltpu.make_async_copy(x_hbm.at[pl.ds(0, CH)], xbuf.at[cur],
                                  semx.at[cur]).wait()

            @pl.when(i + 2 < n_my)
            def _prefetch():
                start_fetch(i + 2, lax.rem(i + 2, NBUF))

            def g_body(g, carry2):
                do_rowgroup(cur, i, g)
                return carry2
            lax.fori_loop(0, CH // LANES, g_body, 0)
            return carry
        lax.fori_loop(0, n_my, chunk_body, 0)

        pltpu.sync_copy(acc, out_hbm.at[wid])

    return pl.kernel(
        body,
        out_type=jax.ShapeDtypeStruct((NW, num_segments, d), jnp.float32),
        mesh=mesh,
        compiler_params=pltpu.CompilerParams(needs_layout_passes=False),
        scratch_types=[
            pltpu.VMEM((num_segments, d), jnp.float32),
            pltpu.VMEM((NBUF, CH, d), jnp.float32),
            pltpu.VMEM((pre_rows,), jnp.int32),
            pltpu.SemaphoreType.DMA((NBUF,)),
            pltpu.SemaphoreType.DMA,
        ],
    )


def _make_mlp(num_segments, d):
    def mlp_body(parts_ref, u_ref, w1_ref, b1_ref, w2_ref, b2_ref, out_ref):
        agg = parts_ref[0]
        for i in range(1, NW):
            agg = jnp.maximum(agg, parts_ref[i])
        agg = jnp.where(jnp.isfinite(agg), agg, 0.0)
        w1 = w1_ref[...]
        h = (jnp.dot(u_ref[...], w1[:d], preferred_element_type=jnp.float32)
             + jnp.dot(agg, w1[d:], preferred_element_type=jnp.float32)
             + b1_ref[...])
        h = jnp.maximum(h, 0.0)
        out_ref[...] = (jnp.dot(h, w2_ref[...],
                                preferred_element_type=jnp.float32)
                        + b2_ref[...])

    return pl.pallas_call(
        mlp_body,
        out_shape=jax.ShapeDtypeStruct((num_segments, d), jnp.float32),
    )


@jax.jit
def kernel(x, batch, u, W1, b1, W2, b2):
    n, d = x.shape
    num_segments = u.shape[0]
    batch = batch.astype(jnp.int32)
    parts = _make_segmax(n, num_segments, d)(x, batch)
    return _make_mlp(num_segments, d)(parts, u, W1, b1.reshape(1, d), W2,
                                      b2.reshape(1, d))


# speedup vs baseline: 1.2888x; 1.0665x over previous
import functools

import jax
import jax.numpy as jnp
from jax import lax
from jax.experimental import pallas as pl
from jax.experimental.pallas import tpu as pltpu
from jax.experimental.pallas import tpu_sc as plsc

NC = 2
NS = 16
NW = NC * NS
LANES = 16
CH = 160
NBUF = 3


def _tree_max(vals):
    while len(vals) > 1:
        nxt = [jnp.maximum(vals[i], vals[i + 1]) for i in range(0, len(vals) - 1, 2)]
        if len(vals) % 2:
            nxt.append(vals[-1])
        vals = nxt
    return vals[0]


def _make_segmax(n_rows, num_segments, d):
    ncg = d // LANES
    n_chunks = n_rows // CH
    assert n_chunks * CH == n_rows
    max_my = -(-n_chunks // NW)
    pre_rows = max_my * CH
    mesh = plsc.VectorSubcoreMesh(core_axis_name="c", subcore_axis_name="s",
                                  num_cores=NC, num_subcores=NS)

    def body(x_hbm, b_hbm, out_hbm, acc, xbuf, bbuf, semx, semb):
        wid = lax.axis_index("s") * NC + lax.axis_index("c")

        start_c = (wid * n_chunks) // NW
        end_c = ((wid + 1) * n_chunks) // NW
        n_my = end_c - start_c

        def start_fetch(i, slot):
            c = start_c + i
            pltpu.async_copy(x_hbm.at[pl.ds(c * CH, CH)], xbuf.at[slot],
                             semx.at[slot])

        pltpu.async_copy(b_hbm.at[pl.ds(start_c * CH, pre_rows)], bbuf, semb)

        @pl.when(n_my > 0)
        def _prime0():
            start_fetch(0, 0)

        @pl.when(n_my > 1)
        def _prime1():
            start_fetch(1, 1)

        neg = jnp.full((LANES,), -jnp.inf, jnp.float32)
        def init_body(i, carry):
            for c in range(ncg):
                acc[i, pl.ds(c * LANES, LANES)] = neg
            return carry
        lax.fori_loop(0, num_segments, init_body, 0)

        pltpu.make_async_copy(b_hbm.at[pl.ds(0, pre_rows)], bbuf, semb).wait()

        def do_rowgroup(par, i, g):
            base = g * LANES
            bvec = bbuf[pl.ds(i * CH + base, LANES)]
            s0 = bvec[0]
            s15 = bvec[LANES - 1]

            @pl.when(s0 == s15)
            def _fast():
                for c in range(ncg):
                    cs = pl.ds(c * LANES, LANES)
                    m = xbuf[par, base, cs]
                    for r in range(1, LANES):
                        m = jnp.maximum(m, xbuf[par, base + r, cs])
                    acc[s0, cs] = jnp.maximum(acc[s0, cs], m)

            @pl.when(s0 != s15)
            def _boundary():
                sv0 = jnp.full((LANES,), s0, jnp.int32)
                sv15 = jnp.full((LANES,), s15, jnp.int32)
                both = (bvec == sv0) | (bvec == sv15)
                n2 = plsc.all_reduce_population_count(both)[0]

                @pl.when(n2 == LANES)
                def _two_seg():
                    neginf = jnp.full((LANES,), -jnp.inf, jnp.float32)
                    conds = [bvec[r] == s0 for r in range(LANES)]
                    for c in range(ncg):
                        cs = pl.ds(c * LANES, LANES)
                        v = xbuf[par, base, cs]
                        m0 = jnp.where(conds[0], v, neginf)
                        m1 = jnp.where(conds[0], neginf, v)
                        for r in range(1, LANES):
                            v = xbuf[par, base + r, cs]
                            m0 = jnp.maximum(m0, jnp.where(conds[r], v, neginf))
                            m1 = jnp.maximum(m1, jnp.where(conds[r], neginf, v))
                        acc[s0, cs] = jnp.maximum(acc[s0, cs], m0)
                        acc[s15, cs] = jnp.maximum(acc[s15, cs], m1)

                @pl.when(n2 != LANES)
                def _general():
                    for j in range(LANES):
                        sj = bvec[j]
                        for c in range(ncg):
                            cs = pl.ds(c * LANES, LANES)
                            v = xbuf[par, base + j, cs]
                            acc[sj, cs] = jnp.maximum(acc[sj, cs], v)

        def chunk_body(i, carry):
            cur = lax.rem(i, NBUF)
            pltpu.make_async_copy(x_hbm.at[pl.ds(0, CH)], xbuf.at[cur],
                                  semx.at[cur]).wait()

            @pl.when(i + 2 < n_my)
            def _prefetch():
                start_fetch(i + 2, lax.rem(i + 2, NBUF))

            def g_body(g, carry2):
                do_rowgroup(cur, i, g)
                return carry2
            lax.fori_loop(0, CH // LANES, g_body, 0)
            return carry
        lax.fori_loop(0, n_my, chunk_body, 0)

        pltpu.sync_copy(acc, out_hbm.at[wid])

    return pl.kernel(
        body,
        out_type=jax.ShapeDtypeStruct((NW, num_segments, d), jnp.float32),
        mesh=mesh,
        compiler_params=pltpu.CompilerParams(needs_layout_passes=False),
        scratch_types=[
            pltpu.VMEM((num_segments, d), jnp.float32),
            pltpu.VMEM((NBUF, CH, d), jnp.float32),
            pltpu.VMEM((pre_rows,), jnp.int32),
            pltpu.SemaphoreType.DMA((NBUF,)),
            pltpu.SemaphoreType.DMA,
        ],
    )


def _make_mlp(num_segments, d):
    def mlp_body(parts_ref, u_ref, w1_ref, b1_ref, w2_ref, b2_ref, out_ref):
        agg = parts_ref[0]
        for i in range(1, NW):
            agg = jnp.maximum(agg, parts_ref[i])
        agg = jnp.where(jnp.isfinite(agg), agg, 0.0)
        w1 = w1_ref[...]
        h = (jnp.dot(u_ref[...], w1[:d], preferred_element_type=jnp.float32)
             + jnp.dot(agg, w1[d:], preferred_element_type=jnp.float32)
             + b1_ref[...])
        h = jnp.maximum(h, 0.0)
        out_ref[...] = (jnp.dot(h, w2_ref[...],
                                preferred_element_type=jnp.float32)
                        + b2_ref[...])

    return pl.pallas_call(
        mlp_body,
        out_shape=jax.ShapeDtypeStruct((num_segments, d), jnp.float32),
    )


@jax.jit
def kernel(x, batch, u, W1, b1, W2, b2):
    n, d = x.shape
    num_segments = u.shape[0]
    batch = batch.astype(jnp.int32)
    parts = _make_segmax(n, num_segments, d)(x, batch)
    return _make_mlp(num_segments, d)(parts, u, W1, b1.reshape(1, d), W2,
                                      b2.reshape(1, d))
